# baseline (speedup 1.0000x reference)
import functools

import jax
import jax.numpy as jnp
from jax import lax
from jax.experimental import pallas as pl
from jax.experimental.pallas import tpu as pltpu
from jax.experimental.pallas import tpu_sc as plsc

CODEBOOK = 1024
DIM = 64
M_BLK = 1024


def _vq_body(flat_ref, emb_ref, e2_ref, f2_ref, colf_ref, idx_ref, loss_ref,
             *, n_total):
    step = pl.program_id(0)
    flat = flat_ref[...]
    emb = emb_ref[...]

    dotm2 = jax.lax.dot_general(
        flat * -2.0, emb,
        dimension_numbers=(((1,), (1,)), ((), ())),
        preferred_element_type=jnp.float32,
    )
    d = (f2_ref[...] + dotm2) + e2_ref[...]

    dmin = jnp.min(d, axis=1, keepdims=True)
    colf = colf_ref[...]
    idxf = jnp.min(jnp.where(d == dmin, colf, 2048.0), axis=1, keepdims=True)
    idx_ref[...] = idxf.astype(jnp.int32)

    part = jnp.sum(dmin).reshape(1, 1)

    @pl.when(step == 0)
    def _():
        loss_ref[...] = jnp.zeros((1, 1), jnp.float32)

    loss_ref[...] += part

    @pl.when(step == pl.num_programs(0) - 1)
    def _():
        loss_ref[...] = loss_ref[...] / n_total


def _make_sc_gather(n, d, n_workers, num_cores):

    b_per_w = n // n_workers
    mesh = plsc.VectorSubcoreMesh(core_axis_name="c", subcore_axis_name="s")

    @functools.partial(
        pl.kernel, mesh=mesh,
        out_type=jax.ShapeDtypeStruct((n, d), jnp.float32),
        scratch_types=[
            pltpu.VMEM((b_per_w,), jnp.int32),
            pltpu.VMEM((b_per_w, d), jnp.float32),
            pltpu.SemaphoreType.DMA,
        ],
    )
    def gather_rows(table_hbm, idx_hbm, out_hbm, idx_v, rows_v, sem):
        wid = lax.axis_index("s") * num_cores + lax.axis_index("c")
        base = wid * b_per_w
        pltpu.sync_copy(idx_hbm.at[pl.ds(base, b_per_w)], idx_v)
        pltpu.async_copy(table_hbm.at[idx_v], rows_v, sem).wait()
        pltpu.sync_copy(rows_v, out_hbm.at[pl.ds(base, b_per_w)])

    return gather_rows


def kernel(inputs, embedding):
    B, T, D = inputs.shape
    n = B * T
    flat = inputs.reshape(n, D)
    grid = n // M_BLK

    idx, loss = pl.pallas_call(
        functools.partial(_vq_body, n_total=float(n * D)),
        grid=(grid,),
        in_specs=[
            pl.BlockSpec((M_BLK, D), lambda i: (i, 0)),
            pl.BlockSpec((CODEBOOK, D), lambda i: (0, 0)),
            pl.BlockSpec((1, CODEBOOK), lambda i: (0, 0)),
            pl.BlockSpec((M_BLK, 1), lambda i: (i, 0)),
            pl.BlockSpec((1, CODEBOOK), lambda i: (0, 0)),
        ],
        out_specs=[
            pl.BlockSpec((M_BLK, 1), lambda i: (i, 0)),
            pl.BlockSpec((1, 1), lambda i: (0, 0)),
        ],
        out_shape=[
            jax.ShapeDtypeStruct((n, 1), jnp.int32),
            jax.ShapeDtypeStruct((1, 1), jnp.float32),
        ],
    )(flat, embedding, jnp.sum(embedding**2, axis=1)[None, :],
      jnp.sum(flat**2, axis=1, keepdims=True),
      jnp.arange(CODEBOOK, dtype=jnp.float32)[None, :])

    info = plsc.get_sparse_core_info()
    n_workers = info.num_cores * info.num_subcores
    idx_flat = idx.reshape(n)
    emb128 = jnp.concatenate(
        [embedding, jnp.zeros((CODEBOOK, 128 - D), jnp.float32)], axis=1)
    out128 = _make_sc_gather(n, 128, n_workers, info.num_cores)(emb128, idx_flat)
    qst = out128[:, :D]

    return (qst.reshape(inputs.shape),
            idx.reshape(B, T),
            loss[0, 0])


# speedup vs baseline: 1.3496x; 1.3496x over previous
import functools

import jax
import jax.numpy as jnp
from jax import lax
from jax.experimental import pallas as pl
from jax.experimental.pallas import tpu as pltpu
from jax.experimental.pallas import tpu_sc as plsc

CODEBOOK = 1024
DIM = 64
M_BLK = 1024


def _vq_body(flat_ref, emb_ref, e2_ref, f2_ref, colf_ref, qst_ref, idx_ref,
             loss_ref, *, n_total):
    step = pl.program_id(0)
    flat = flat_ref[...]
    emb = emb_ref[...]

    dotm2 = jax.lax.dot_general(
        flat * -2.0, emb,
        dimension_numbers=(((1,), (1,)), ((), ())),
        preferred_element_type=jnp.float32,
    )
    d = (f2_ref[...] + dotm2) + e2_ref[...]

    dmin = jnp.min(d, axis=1, keepdims=True)
    colf = colf_ref[...]
    idxf = jnp.min(jnp.where(d == dmin, colf, 2048.0), axis=1, keepdims=True)
    idx_ref[...] = idxf.astype(jnp.int32)

    onehot = (colf == idxf).astype(jnp.float32)
    qst_ref[...] = jax.lax.dot_general(
        onehot, emb,
        dimension_numbers=(((1,), (0,)), ((), ())),
        preferred_element_type=jnp.float32,
    )

    part = jnp.sum(dmin).reshape(1, 1)

    @pl.when(step == 0)
    def _():
        loss_ref[...] = jnp.zeros((1, 1), jnp.float32)

    loss_ref[...] += part

    @pl.when(step == pl.num_programs(0) - 1)
    def _():
        loss_ref[...] = loss_ref[...] / n_total


def _make_sc_gather(n, d, n_workers, num_cores):

    b_per_w = n // n_workers
    mesh = plsc.VectorSubcoreMesh(core_axis_name="c", subcore_axis_name="s")

    @functools.partial(
        pl.kernel, mesh=mesh,
        out_type=jax.ShapeDtypeStruct((n, d), jnp.float32),
        scratch_types=[
            pltpu.VMEM((b_per_w,), jnp.int32),
            pltpu.VMEM((b_per_w, d), jnp.float32),
            pltpu.SemaphoreType.DMA,
        ],
    )
    def gather_rows(table_hbm, idx_hbm, out_hbm, idx_v, rows_v, sem):
        wid = lax.axis_index("s") * num_cores + lax.axis_index("c")
        base = wid * b_per_w
        pltpu.sync_copy(idx_hbm.at[pl.ds(base, b_per_w)], idx_v)
        pltpu.async_copy(table_hbm.at[idx_v], rows_v, sem).wait()
        pltpu.sync_copy(rows_v, out_hbm.at[pl.ds(base, b_per_w)])

    return gather_rows


def kernel(inputs, embedding):
    B, T, D = inputs.shape
    n = B * T
    flat = inputs.reshape(n, D)
    grid = n // M_BLK

    qst, idx, loss = pl.pallas_call(
        functools.partial(_vq_body, n_total=float(n * D)),
        grid=(grid,),
        in_specs=[
            pl.BlockSpec((M_BLK, D), lambda i: (i, 0)),
            pl.BlockSpec((CODEBOOK, D), lambda i: (0, 0)),
            pl.BlockSpec((1, CODEBOOK), lambda i: (0, 0)),
            pl.BlockSpec((M_BLK, 1), lambda i: (i, 0)),
            pl.BlockSpec((1, CODEBOOK), lambda i: (0, 0)),
        ],
        out_specs=[
            pl.BlockSpec((M_BLK, D), lambda i: (i, 0)),
            pl.BlockSpec((M_BLK, 1), lambda i: (i, 0)),
            pl.BlockSpec((1, 1), lambda i: (0, 0)),
        ],
        out_shape=[
            jax.ShapeDtypeStruct((n, D), jnp.float32),
            jax.ShapeDtypeStruct((n, 1), jnp.int32),
            jax.ShapeDtypeStruct((1, 1), jnp.float32),
        ],
    )(flat, embedding, jnp.sum(embedding**2, axis=1)[None, :],
      jnp.sum(flat**2, axis=1, keepdims=True),
      jnp.arange(CODEBOOK, dtype=jnp.float32)[None, :])

    return (qst.reshape(inputs.shape),
            idx.reshape(B, T),
            loss[0, 0])


# speedup vs baseline: 1.5352x; 1.1375x over previous
import functools

import jax
import jax.numpy as jnp
from jax import lax
from jax.experimental import pallas as pl
from jax.experimental.pallas import tpu as pltpu
from jax.experimental.pallas import tpu_sc as plsc

CODEBOOK = 1024
DIM = 64
M_BLK = 1024


def _vq_body(flat_ref, emb_ref, e2_ref, colf_ref, qst_ref, idx_ref,
             loss_ref, *, n_total):
    step = pl.program_id(0)
    flat = flat_ref[...]
    emb = emb_ref[...]

    dotm2 = jax.lax.dot_general(
        flat * -2.0, emb,
        dimension_numbers=(((1,), (1,)), ((), ())),
        preferred_element_type=jnp.float32,
    )
    f2 = jnp.sum(flat * flat, axis=1, keepdims=True)
    d = (f2 + dotm2) + e2_ref[...]

    dmin = jnp.min(d, axis=1, keepdims=True)
    colf = colf_ref[...]
    idxf = jnp.min(jnp.where(d == dmin, colf, 2048.0), axis=1, keepdims=True)
    idx_ref[...] = idxf.astype(jnp.int32)

    onehot = (colf == idxf).astype(jnp.float32)
    qst_ref[...] = jax.lax.dot_general(
        onehot, emb,
        dimension_numbers=(((1,), (0,)), ((), ())),
        preferred_element_type=jnp.float32,
    )

    part = jnp.sum(dmin).reshape(1, 1)

    @pl.when(step == 0)
    def _():
        loss_ref[...] = jnp.zeros((1, 1), jnp.float32)

    loss_ref[...] += part

    @pl.when(step == pl.num_programs(0) - 1)
    def _():
        loss_ref[...] = loss_ref[...] / n_total


def _make_sc_gather(n, d, n_workers, num_cores):

    b_per_w = n // n_workers
    mesh = plsc.VectorSubcoreMesh(core_axis_name="c", subcore_axis_name="s")

    @functools.partial(
        pl.kernel, mesh=mesh,
        out_type=jax.ShapeDtypeStruct((n, d), jnp.float32),
        scratch_types=[
            pltpu.VMEM((b_per_w,), jnp.int32),
            pltpu.VMEM((b_per_w, d), jnp.float32),
            pltpu.SemaphoreType.DMA,
        ],
    )
    def gather_rows(table_hbm, idx_hbm, out_hbm, idx_v, rows_v, sem):
        wid = lax.axis_index("s") * num_cores + lax.axis_index("c")
        base = wid * b_per_w
        pltpu.sync_copy(idx_hbm.at[pl.ds(base, b_per_w)], idx_v)
        pltpu.async_copy(table_hbm.at[idx_v], rows_v, sem).wait()
        pltpu.sync_copy(rows_v, out_hbm.at[pl.ds(base, b_per_w)])

    return gather_rows


def kernel(inputs, embedding):
    B, T, D = inputs.shape
    n = B * T
    flat = inputs.reshape(n, D)
    grid = n // M_BLK

    qst, idx, loss = pl.pallas_call(
        functools.partial(_vq_body, n_total=float(n * D)),
        grid=(grid,),
        in_specs=[
            pl.BlockSpec((M_BLK, D), lambda i: (i, 0)),
            pl.BlockSpec((CODEBOOK, D), lambda i: (0, 0)),
            pl.BlockSpec((1, CODEBOOK), lambda i: (0, 0)),
            pl.BlockSpec((1, CODEBOOK), lambda i: (0, 0)),
        ],
        out_specs=[
            pl.BlockSpec((M_BLK, D), lambda i: (i, 0)),
            pl.BlockSpec((M_BLK, 1), lambda i: (i, 0)),
            pl.BlockSpec((1, 1), lambda i: (0, 0)),
        ],
        out_shape=[
            jax.ShapeDtypeStruct((n, D), jnp.float32),
            jax.ShapeDtypeStruct((n, 1), jnp.int32),
            jax.ShapeDtypeStruct((1, 1), jnp.float32),
        ],
    )(flat, embedding, jnp.sum(embedding**2, axis=1)[None, :],
      jnp.arange(CODEBOOK, dtype=jnp.float32)[None, :])

    return (qst.reshape(inputs.shape),
            idx.reshape(B, T),
            loss[0, 0])


# speedup vs baseline: 1.6006x; 1.0426x over previous
import functools

import jax
import jax.numpy as jnp
from jax import lax
from jax.experimental import pallas as pl
from jax.experimental.pallas import tpu as pltpu
from jax.experimental.pallas import tpu_sc as plsc

CODEBOOK = 1024
DIM = 64
M_BLK = 1024


def _vq_body(flat_ref, emb_ref, e2_ref, colf_ref, qst_ref, idx_ref,
             loss_ref, *, n_total):
    step = pl.program_id(0)
    flat = flat_ref[...]
    emb = emb_ref[...]

    dotm2 = jax.lax.dot_general(
        flat * -2.0, emb,
        dimension_numbers=(((1,), (1,)), ((), ())),
        preferred_element_type=jnp.float32,
    )
    f2 = jnp.sum(flat * flat, axis=1, keepdims=True)
    d = (f2 + dotm2) + e2_ref[...]

    dmin = jnp.min(d, axis=1, keepdims=True)
    colf = colf_ref[...]
    idxf = jnp.min(jnp.where(d == dmin, colf, 2048.0), axis=1, keepdims=True)
    idx_ref[...] = idxf.astype(jnp.int32).reshape(1, 8, 128)

    onehot = (colf == idxf).astype(jnp.float32)
    qst_ref[...] = jax.lax.dot_general(
        onehot, emb,
        dimension_numbers=(((1,), (0,)), ((), ())),
        preferred_element_type=jnp.float32,
    )

    part = jnp.sum(dmin).reshape(1, 1)

    @pl.when(step == 0)
    def _():
        loss_ref[...] = jnp.zeros((1, 1), jnp.float32)

    loss_ref[...] += part

    @pl.when(step == pl.num_programs(0) - 1)
    def _():
        loss_ref[...] = loss_ref[...] / n_total


def _make_sc_gather(n, d, n_workers, num_cores):

    b_per_w = n // n_workers
    mesh = plsc.VectorSubcoreMesh(core_axis_name="c", subcore_axis_name="s")

    @functools.partial(
        pl.kernel, mesh=mesh,
        out_type=jax.ShapeDtypeStruct((n, d), jnp.float32),
        scratch_types=[
            pltpu.VMEM((b_per_w,), jnp.int32),
            pltpu.VMEM((b_per_w, d), jnp.float32),
            pltpu.SemaphoreType.DMA,
        ],
    )
    def gather_rows(table_hbm, idx_hbm, out_hbm, idx_v, rows_v, sem):
        wid = lax.axis_index("s") * num_cores + lax.axis_index("c")
        base = wid * b_per_w
        pltpu.sync_copy(idx_hbm.at[pl.ds(base, b_per_w)], idx_v)
        pltpu.async_copy(table_hbm.at[idx_v], rows_v, sem).wait()
        pltpu.sync_copy(rows_v, out_hbm.at[pl.ds(base, b_per_w)])

    return gather_rows


def kernel(inputs, embedding):
    B, T, D = inputs.shape
    n = B * T
    flat = inputs.reshape(n, D)
    grid = n // M_BLK

    qst, idx, loss = pl.pallas_call(
        functools.partial(_vq_body, n_total=float(n * D)),
        grid=(grid,),
        in_specs=[
            pl.BlockSpec((M_BLK, D), lambda i: (i, 0)),
            pl.BlockSpec((CODEBOOK, D), lambda i: (0, 0)),
            pl.BlockSpec((1, CODEBOOK), lambda i: (0, 0)),
            pl.BlockSpec((1, CODEBOOK), lambda i: (0, 0)),
        ],
        out_specs=[
            pl.BlockSpec((M_BLK, D), lambda i: (i, 0)),
            pl.BlockSpec((1, 8, 128), lambda i: (i, 0, 0)),
            pl.BlockSpec((1, 1), lambda i: (0, 0)),
        ],
        out_shape=[
            jax.ShapeDtypeStruct((n, D), jnp.float32),
            jax.ShapeDtypeStruct((grid, 8, 128), jnp.int32),
            jax.ShapeDtypeStruct((1, 1), jnp.float32),
        ],
    )(flat, embedding, jnp.sum(embedding**2, axis=1)[None, :],
      jnp.arange(CODEBOOK, dtype=jnp.float32)[None, :])

    return (qst.reshape(inputs.shape),
            idx.reshape(B, T),
            loss[0, 0])
